# initial kernel scaffold (unmeasured)
import jax
import jax.numpy as jnp
from jax import lax
from jax.experimental import pallas as pl
from jax.experimental.pallas import tpu as pltpu


def kernel(
    x,
):
    def body(*refs):
        pass

    out_shape = jax.ShapeDtypeStruct(..., jnp.float32)
    return pl.pallas_call(body, out_shape=out_shape)(...)


# baseline (device time: 17389 ns/iter reference)
import jax
import jax.numpy as jnp
from jax import lax
from jax.experimental import pallas as pl
from jax.experimental.pallas import tpu as pltpu

N_DEV = 32
DISTS = (1, 1, 2, 4, 8, 16)


def kernel(x):
    m, n = x.shape

    def body(x_ref, out_ref, comm_ref, send_ref, send_sems, recv_sems):
        my = lax.axis_index("i")

        y = x_ref[:, :].astype(jnp.float32)
        row = lax.broadcasted_iota(jnp.int32, (m, n), 0)
        s = 1
        while s < m:
            shifted = pltpu.roll(y, s, 0)
            y = y * jnp.where(row >= s, shifted, 1.0)
            s *= 2
        total = y[m - 1 : m, :]

        acc = None
        for r, d in enumerate(DISTS):
            snap = total if r == 0 else acc
            send_ref[r, :, :] = snap

            def _mk(r=r, d=d):
                return pltpu.make_async_remote_copy(
                    src_ref=send_ref.at[r],
                    dst_ref=comm_ref.at[r],
                    send_sem=send_sems.at[r],
                    recv_sem=recv_sems.at[r],
                    device_id=(lax.rem(my + d, N_DEV),),
                    device_id_type=pl.DeviceIdType.MESH,
                )

            @pl.when(my + d < N_DEV)
            def _send(_mk=_mk):
                rdma = _mk()
                rdma.start()
                rdma.wait_send()

            @pl.when(my >= d)
            def _recv(_mk=_mk):
                _mk().wait_recv()

            rcvd = jnp.where(my >= d, comm_ref[r, :, :], 1.0)
            acc = rcvd if r == 0 else acc * rcvd

        out_ref[:, :] = y * acc

    return pl.pallas_call(
        body,
        out_shape=jax.ShapeDtypeStruct((m, n), jnp.float32),
        in_specs=[pl.BlockSpec(memory_space=pltpu.VMEM)],
        out_specs=pl.BlockSpec(memory_space=pltpu.VMEM),
        scratch_shapes=[
            pltpu.VMEM((len(DISTS), 1, n), jnp.float32),
            pltpu.VMEM((len(DISTS), 1, n), jnp.float32),
            pltpu.SemaphoreType.DMA((len(DISTS),)),
            pltpu.SemaphoreType.DMA((len(DISTS),)),
        ],
    )(x)


# device time: 13406 ns/iter; 1.2971x vs baseline; 1.2971x over previous
import jax
import jax.numpy as jnp
from jax import lax
from jax.experimental import pallas as pl
from jax.experimental.pallas import tpu as pltpu

N_DEV = 32


def kernel(x):
    m, n = x.shape
    nd = N_DEV - 1

    def body(x_ref, out_ref, comm_ref, send_ref, send_sems, recv_sems):
        my = lax.axis_index("i")
        xf = x_ref[:, :].astype(jnp.float32)

        t = xf
        while t.shape[0] > 1:
            h = t.shape[0] // 2
            t = t[:h, :] * t[h:, :]
        send_ref[:, :] = t

        def _mk(d):
            return pltpu.make_async_remote_copy(
                src_ref=send_ref,
                dst_ref=comm_ref.at[d - 1],
                send_sem=send_sems.at[d - 1],
                recv_sem=recv_sems.at[d - 1],
                device_id=(lax.rem(my + d, N_DEV),),
                device_id_type=pl.DeviceIdType.MESH,
            )

        for d in range(1, N_DEV):

            @pl.when(my + d < N_DEV)
            def _send(d=d):
                _mk(d).start()

        row = lax.broadcasted_iota(jnp.int32, (m, n), 0)
        y = xf
        s = 1
        while s < m:
            shifted = pltpu.roll(y, s, 0)
            y = y * jnp.where(row >= s, shifted, 1.0)
            s *= 2

        for d in range(1, N_DEV):

            @pl.when(my >= d)
            def _recv(d=d):
                _mk(d).wait_recv()

        dist = lax.broadcasted_iota(jnp.int32, (nd, n), 0)
        vals = jnp.where(dist < my, comm_ref[:, 0, :], 1.0)
        rows = [vals[i : i + 1, :] for i in range(nd)]
        while len(rows) > 1:
            rows = [
                rows[i] * rows[i + 1] if i + 1 < len(rows) else rows[i]
                for i in range(0, len(rows), 2)
            ]
        excl = rows[0]

        out_ref[:, :] = y * excl

        for d in range(1, N_DEV):

            @pl.when(my + d < N_DEV)
            def _drain(d=d):
                _mk(d).wait_send()

    return pl.pallas_call(
        body,
        out_shape=jax.ShapeDtypeStruct((m, n), jnp.float32),
        in_specs=[pl.BlockSpec(memory_space=pltpu.VMEM)],
        out_specs=pl.BlockSpec(memory_space=pltpu.VMEM),
        scratch_shapes=[
            pltpu.VMEM((nd, 1, n), jnp.float32),
            pltpu.VMEM((1, n), jnp.float32),
            pltpu.SemaphoreType.DMA((nd,)),
            pltpu.SemaphoreType.DMA((nd,)),
        ],
    )(x)


# device time: 11634 ns/iter; 1.4947x vs baseline; 1.1523x over previous
import jax
import jax.numpy as jnp
from jax import lax
from jax.experimental import pallas as pl
from jax.experimental.pallas import tpu as pltpu

N_DEV = 32


def kernel(x):
    m, n = x.shape
    nd = N_DEV - 1

    def body(x_ref, out_ref, comm_ref, send_ref, send_sems, recv_sems):
        my = lax.axis_index("i")

        bar = pltpu.get_barrier_semaphore()
        for k in range(N_DEV):

            @pl.when(my != k)
            def _(k=k):
                pl.semaphore_signal(
                    bar, inc=1, device_id=(k,),
                    device_id_type=pl.DeviceIdType.MESH,
                )

        xf = x_ref[:, :].astype(jnp.float32)

        t = xf
        while t.shape[0] > 1:
            h = t.shape[0] // 2
            t = t[:h, :] * t[h:, :]
        send_ref[:, :] = t

        row = lax.broadcasted_iota(jnp.int32, (m, n), 0)
        y = xf
        s = 1
        while s < m:
            shifted = pltpu.roll(y, s, 0)
            y = y * jnp.where(row >= s, shifted, 1.0)
            s *= 2

        pl.semaphore_wait(bar, N_DEV - 1)

        def _mk(d):
            return pltpu.make_async_remote_copy(
                src_ref=send_ref,
                dst_ref=comm_ref.at[d - 1],
                send_sem=send_sems.at[d - 1],
                recv_sem=recv_sems.at[d - 1],
                device_id=(lax.rem(my + d, N_DEV),),
                device_id_type=pl.DeviceIdType.MESH,
            )

        for d in range(1, N_DEV):

            @pl.when(my + d < N_DEV)
            def _send(d=d):
                _mk(d).start()

        for d in range(1, N_DEV):

            @pl.when(my >= d)
            def _recv(d=d):
                _mk(d).wait_recv()

        dist = lax.broadcasted_iota(jnp.int32, (nd, n), 0)
        vals = jnp.where(dist < my, comm_ref[:, 0, :], 1.0)
        rows = [vals[i : i + 1, :] for i in range(nd)]
        while len(rows) > 1:
            rows = [
                rows[i] * rows[i + 1] if i + 1 < len(rows) else rows[i]
                for i in range(0, len(rows), 2)
            ]

        out_ref[:, :] = y * rows[0]

        for d in range(1, N_DEV):

            @pl.when(my + d < N_DEV)
            def _drain(d=d):
                _mk(d).wait_send()

    return pl.pallas_call(
        body,
        out_shape=jax.ShapeDtypeStruct((m, n), jnp.float32),
        in_specs=[pl.BlockSpec(memory_space=pltpu.VMEM)],
        out_specs=pl.BlockSpec(memory_space=pltpu.VMEM),
        scratch_shapes=[
            pltpu.VMEM((nd, 1, n), jnp.float32),
            pltpu.VMEM((1, n), jnp.float32),
            pltpu.SemaphoreType.DMA((nd,)),
            pltpu.SemaphoreType.DMA((nd,)),
        ],
        compiler_params=pltpu.CompilerParams(collective_id=0),
    )(x)
